# Initial kernel scaffold; baseline (speedup 1.0000x reference)
import jax
import jax.numpy as jnp
from jax.experimental import pallas as pl


def kernel(x, W_qk, b_qk, W_v, b_v, W_out, b_out):
    raise NotImplementedError("write your pallas kernel here")


# speedup vs baseline: 1.4764x; 1.4764x over previous
import functools
import math

import jax
import jax.numpy as jnp
from jax.experimental import pallas as pl

N_EMBD = 768
N_HEAD = 12
HEAD_DIM = N_EMBD // N_HEAD
T = 2048
SCALE = 1.0 / math.sqrt(HEAD_DIM)
STRIDES = (1, 3, 7, 13)

BQ = 256
BR = 256


def _proj2_body(x_ref, wqk_ref, bqk_ref, wv_ref, bv_ref, qk_ref, v_ref):
    xb = x_ref[:]
    qk_ref[:] = jax.lax.dot_general(
        xb, wqk_ref[:], (((1,), (1,)), ((), ())),
        preferred_element_type=jnp.float32) + bqk_ref[:]
    v_ref[:] = jax.lax.dot_general(
        xb, wv_ref[:], (((1,), (1,)), ((), ())),
        preferred_element_type=jnp.float32) + bv_ref[:]


def _attn_body(qk_ref, v_ref, wout_ref, bout_ref, o_ref):
    qi = pl.program_id(0)
    q_all = qk_ref[pl.ds(qi * BQ, BQ), :]

    i = qi * BQ + jax.lax.broadcasted_iota(jnp.int32, (BQ, T), 0)
    p = jax.lax.broadcasted_iota(jnp.int32, (BQ, T), 1)
    d = (p - i) & (T - 1)
    causal = p <= i
    half = d < (T // 2)

    masks = []
    for s in STRIDES:
        if s == 1:
            masks.append(causal)
        else:
            rA = (-(T // 2)) % s
            rB = (T // 2) % s
            ds = d % s
            masks.append(causal & ((half & (ds == rA)) | (~half & (ds == rB))))

    outs = []
    for h in range(N_HEAD):
        q = q_all[:, h * HEAD_DIM:(h + 1) * HEAD_DIM]
        k = qk_ref[:, h * HEAD_DIM:(h + 1) * HEAD_DIM]
        v = v_ref[:, h * HEAD_DIM:(h + 1) * HEAD_DIM]
        logits = jax.lax.dot_general(
            q, k, (((1,), (1,)), ((), ())),
            preferred_element_type=jnp.float32) * SCALE
        valid = masks[h % 4]
        masked = jnp.where(valid, logits, -jnp.inf)
        m = jnp.max(masked, axis=-1, keepdims=True)
        m_safe = jnp.where(jnp.isfinite(m), m, 0.0)
        e = jnp.where(valid, jnp.exp(masked - m_safe), 0.0)
        denom = jnp.sum(e, axis=-1, keepdims=True)
        w = jnp.where(denom > 0, e / jnp.maximum(denom, 1e-30), 0.0)
        outs.append(jax.lax.dot_general(
            w, v, (((1,), (0,)), ((), ())),
            preferred_element_type=jnp.float32))

    attn = jnp.concatenate(outs, axis=1)
    o_ref[:] = jax.lax.dot_general(
        attn, wout_ref[:], (((1,), (1,)), ((), ())),
        preferred_element_type=jnp.float32) + bout_ref[:]


@functools.partial(jax.jit, static_argnames=("interpret",))
def _run(x, W_qk, b_qk, W_v, b_v, W_out, b_out, interpret=False):
    x2 = x.reshape(T, N_EMBD)
    nr = T // BR

    qk, vv = pl.pallas_call(
        _proj2_body,
        grid=(nr,),
        in_specs=[
            pl.BlockSpec((BR, N_EMBD), lambda r: (r, 0)),
            pl.BlockSpec((N_EMBD, N_EMBD), lambda r: (0, 0)),
            pl.BlockSpec((N_EMBD,), lambda r: (0,)),
            pl.BlockSpec((N_EMBD, N_EMBD), lambda r: (0, 0)),
            pl.BlockSpec((N_EMBD,), lambda r: (0,)),
        ],
        out_specs=[
            pl.BlockSpec((BR, N_EMBD), lambda r: (r, 0)),
            pl.BlockSpec((BR, N_EMBD), lambda r: (r, 0)),
        ],
        out_shape=[
            jax.ShapeDtypeStruct((T, N_EMBD), jnp.float32),
            jax.ShapeDtypeStruct((T, N_EMBD), jnp.float32),
        ],
        interpret=interpret,
    )(x2, W_qk, b_qk, W_v, b_v)

    nq = T // BQ
    out = pl.pallas_call(
        _attn_body,
        grid=(nq,),
        in_specs=[
            pl.BlockSpec((T, N_EMBD), lambda qi: (0, 0)),
            pl.BlockSpec((T, N_EMBD), lambda qi: (0, 0)),
            pl.BlockSpec((N_EMBD, N_EMBD), lambda qi: (0, 0)),
            pl.BlockSpec((N_EMBD,), lambda qi: (0,)),
        ],
        out_specs=pl.BlockSpec((BQ, N_EMBD), lambda qi: (qi, 0)),
        out_shape=jax.ShapeDtypeStruct((T, N_EMBD), jnp.float32),
        interpret=interpret,
    )(qk, vv, W_out, b_out)

    return out.reshape(1, T, N_EMBD)


def kernel(x, W_qk, b_qk, W_v, b_v, W_out, b_out):
    return _run(x, W_qk, b_qk, W_v, b_v, W_out, b_out)
